# baseline (speedup 1.0000x reference)
import functools

import jax
import jax.numpy as jnp
from jax import lax
from jax.experimental import pallas as pl
from jax.experimental.pallas import tpu as pltpu
from jax.experimental.pallas import tpu_sc as plsc

B, H, W, C = 8, 224, 224, 96
S = H * W
NW = 32
SP = B * S // NW
R = 112
NCH = SP // R
CG = C // 16


def _seg_body(x_hbm, y_hbm, flat_hbm, out_hbm,
              flat_v, red_f, red_i, xc0, xc1, yc0, yc1, out_v,
              sx0, sx1, sy0, sy1):
    wid = lax.axis_index("s") * 2 + lax.axis_index("c")
    r0 = wid * SP
    b = r0 // S
    iv = lax.iota(jnp.int32, 16)

    pltpu.sync_copy(flat_hbm.at[b], flat_v)
    vals = [flat_v[pl.ds(g * 16, 16)] for g in range(CG)]
    mv = vals[0]
    for g in range(1, CG):
        mv = jnp.maximum(mv, vals[g])
    for sh in (8, 4, 2, 1):
        red_f[...] = mv
        mv = jnp.maximum(mv, plsc.load_gather(red_f, [iv ^ sh]))
    acc = iv * 0 + jnp.int32(C)
    for g in range(CG):
        cand = jnp.where(vals[g] == mv, iv + g * 16, jnp.int32(C))
        acc = jnp.minimum(acc, cand)
    for sh in (8, 4, 2, 1):
        red_i[...] = acc
        acc = jnp.minimum(acc, plsc.load_gather(red_i, [iv ^ sh]))
    lidx = acc

    xc = (xc0, xc1)
    yc = (yc0, yc1)
    sx = (sx0, sx1)
    sy = (sy0, sy1)

    def start(j, p):
        pltpu.async_copy(x_hbm.at[pl.ds(r0 + j * R, R), :], xc[p], sx[p])
        pltpu.async_copy(y_hbm.at[pl.ds(r0 + j * R, R), :], yc[p], sy[p])

    def finish(j, p):
        pltpu.make_async_copy(x_hbm.at[pl.ds(r0, R), :], xc[p], sx[p]).wait()
        pltpu.make_async_copy(y_hbm.at[pl.ds(r0, R), :], yc[p], sy[p]).wait()
        for g in range(R // 16):
            rid = iv + g * 16
            xv = plsc.load_gather(xc[p], [rid, lidx])
            yv = plsc.load_gather(yc[p], [rid, lidx])
            out_v[pl.ds(j * R + g * 16, 16)] = xv + yv

    start(0, 0)
    start(1, 1)

    def step(jj, carry):
        j = jj * 2
        finish(j, 0)
        start(j + 2, 0)
        finish(j + 1, 1)
        start(j + 3, 1)
        return carry
    lax.fori_loop(0, NCH // 2 - 1, step, 0)

    finish(NCH - 2, 0)
    finish(NCH - 1, 1)

    pltpu.sync_copy(out_v, out_hbm.at[pl.ds(r0, SP)])


_seg_gather = functools.partial(
    pl.kernel,
    mesh=plsc.VectorSubcoreMesh(core_axis_name="c", subcore_axis_name="s"),
    out_type=jax.ShapeDtypeStruct((B * S,), jnp.float32),
    compiler_params=pltpu.CompilerParams(
        needs_layout_passes=False, use_tc_tiling_on_sc=True),
    scratch_types=[
        pltpu.VMEM((C,), jnp.float32),
        pltpu.VMEM((16,), jnp.float32),
        pltpu.VMEM((16,), jnp.int32),
        pltpu.VMEM((R, C), jnp.float32),
        pltpu.VMEM((R, C), jnp.float32),
        pltpu.VMEM((R, C), jnp.float32),
        pltpu.VMEM((R, C), jnp.float32),
        pltpu.VMEM((SP,), jnp.float32),
        pltpu.SemaphoreType.DMA,
        pltpu.SemaphoreType.DMA,
        pltpu.SemaphoreType.DMA,
        pltpu.SemaphoreType.DMA,
    ],
)(_seg_body)


def kernel(x, y, flat):
    x2 = x.reshape(B * S, C)
    y2 = y.reshape(B * S, C)
    out = _seg_gather(x2, y2, flat)
    return out.reshape(B, H, W)


# speedup vs baseline: 3.1469x; 3.1469x over previous
import jax
import jax.numpy as jnp
from jax.experimental import pallas as pl
from jax.experimental.pallas import tpu as pltpu

B, H, W, C = 8, 224, 224, 96
HB = 32
HH = HB // 2


def _seg_block(flat_ref, x1_ref, x2_ref, y1_ref, y2_ref, out_ref):
    b = pl.program_id(0)
    f = flat_ref[pl.ds(b, 1), :]
    iot = jax.lax.broadcasted_iota(jnp.int32, (1, C), 1)
    m = jnp.max(f)
    cand = jnp.where(f == m, iot, jnp.int32(C))
    c = jnp.min(cand)
    oh = (iot == c).astype(jnp.float32).reshape(1, 1, C)
    s1 = x1_ref[0] + y1_ref[0]
    s2 = x2_ref[0] + y2_ref[0]
    out_ref[0, 0:HH] = jnp.sum(s1 * oh, axis=-1)
    out_ref[0, HH:HB] = jnp.sum(s2 * oh, axis=-1)


def kernel(x, y, flat):
    grid = (B, H // HB)
    half = pl.BlockSpec((1, HH, W, C), lambda b, i: (b, 2 * i, 0, 0))
    half2 = pl.BlockSpec((1, HH, W, C), lambda b, i: (b, 2 * i + 1, 0, 0))
    out = pl.pallas_call(
        _seg_block,
        grid=grid,
        in_specs=[
            pl.BlockSpec((B, C), lambda b, i: (0, 0)),
            half, half2, half, half2,
        ],
        out_specs=pl.BlockSpec((1, HB, W), lambda b, i: (b, i, 0)),
        out_shape=jax.ShapeDtypeStruct((B, H, W), jnp.float32),
        compiler_params=pltpu.CompilerParams(
            dimension_semantics=("parallel", "arbitrary"),
        ),
    )(flat, x, x, y, y)
    return out


# speedup vs baseline: 3.1874x; 1.0129x over previous
import jax
import jax.numpy as jnp
from jax.experimental import pallas as pl
from jax.experimental.pallas import tpu as pltpu

B, H, W, C = 8, 224, 224, 96
HB = 56


def _seg_block(flat_ref, x_ref, y_ref, out_ref):
    b = pl.program_id(0)
    f = flat_ref[pl.ds(b, 1), :]
    iot = jax.lax.broadcasted_iota(jnp.int32, (1, C), 1)
    m = jnp.max(f)
    cand = jnp.where(f == m, iot, jnp.int32(C))
    c = jnp.min(cand)
    oh = (iot == c).astype(jnp.float32)
    s = x_ref[0] + y_ref[0]
    out_ref[0] = jnp.sum(s * oh.reshape(1, 1, C), axis=-1)


def kernel(x, y, flat):
    grid = (B, H // HB)
    out = pl.pallas_call(
        _seg_block,
        grid=grid,
        in_specs=[
            pl.BlockSpec((B, C), lambda b, i: (0, 0)),
            pl.BlockSpec((1, HB, W, C), lambda b, i: (b, i, 0, 0)),
            pl.BlockSpec((1, HB, W, C), lambda b, i: (b, i, 0, 0)),
        ],
        out_specs=pl.BlockSpec((1, HB, W), lambda b, i: (b, i, 0)),
        out_shape=jax.ShapeDtypeStruct((B, H, W), jnp.float32),
        compiler_params=pltpu.CompilerParams(
            dimension_semantics=("parallel", "arbitrary"),
        ),
    )(flat, x, y)
    return out
